# baseline (speedup 1.0000x reference)
import functools

import jax
import jax.numpy as jnp
from jax import lax
from jax.experimental import pallas as pl
from jax.experimental.pallas import tpu as pltpu
from jax.experimental.pallas import tpu_sc as plsc

N = 10000
E = 320000
D = 128
NSUB = 16
B = 128
CH = 8
NCH = 20
NB = CH * NCH
E_PAD = NSUB * NB * B
NPAD = 10240
R_TILE = NPAD // NSUB


def _sc_segment_sums(x_pad, e3):
    mesh = plsc.VectorSubcoreMesh(core_axis_name="c", subcore_axis_name="s")

    @functools.partial(
        pl.kernel,
        mesh=mesh,
        out_type=(jax.ShapeDtypeStruct((2, NPAD, D), jnp.float32),
                  jax.ShapeDtypeStruct((2, NPAD), jnp.float32)),
        scratch_types=[
            pltpu.VMEM((2, CH, B), jnp.int32),
            pltpu.VMEM((2, CH, B), jnp.int32),
            pltpu.VMEM((B, D), jnp.float32),
            pltpu.VMEM((B,), jnp.float32),
            pltpu.VMEM((R_TILE,), jnp.float32),
            pltpu.VMEM_SHARED((NPAD, D), jnp.float32),
            pltpu.VMEM_SHARED((NPAD,), jnp.float32),
            pltpu.SemaphoreType.DMA((2,)),
            pltpu.SemaphoreType.DMA,
        ],
    )
    def k(x_hbm, e_hbm, out_hbm, deg_hbm,
          ebg, ebs, rows, ones, zed, acc, dacc, semg, semi):
        c = lax.axis_index("c")
        s = lax.axis_index("s")
        zvec = jnp.zeros((16,), jnp.float32)
        ovec = jnp.ones((16,), jnp.float32)

        def zrow(i, carry):
            for kk in range(D // 16):
                rows[i, pl.ds(kk * 16, 16)] = zvec
            return carry
        lax.fori_loop(0, B, zrow, 0)
        for j in range(B // 16):
            ones[pl.ds(j * 16, 16)] = ovec
        def zdeg(i, carry):
            zed[pl.ds(i * 16, 16)] = zvec
            return carry
        lax.fori_loop(0, R_TILE // 16, zdeg, 0)
        rbase = s * R_TILE
        for j in range(R_TILE // B):
            pltpu.sync_copy(rows, acc.at[pl.ds(rbase + j * B, B)])
        pltpu.sync_copy(zed, dacc.at[pl.ds(rbase, R_TILE)])
        plsc.subcore_barrier()

        cbase = s * NCH

        def chunk(i, carry):
            pltpu.sync_copy(e_hbm.at[c, cbase + i], ebg.at[0])
            pltpu.sync_copy(e_hbm.at[1 - c, cbase + i], ebs.at[0])
            for j in range(CH):
                pltpu.async_copy(
                    x_hbm.at[ebg.at[0, j]], rows, semg.at[0]).wait()
                pltpu.sync_copy(rows, acc.at[ebs.at[0, j]], add=True)
                pltpu.sync_copy(ones, dacc.at[ebs.at[0, j]], add=True)
            return carry

        lax.fori_loop(0, NCH, chunk, 0)
        plsc.subcore_barrier()

        pltpu.sync_copy(acc.at[pl.ds(rbase, R_TILE)],
                        out_hbm.at[c, pl.ds(rbase, R_TILE)])
        pltpu.sync_copy(dacc.at[pl.ds(rbase, R_TILE)],
                        deg_hbm.at[c, pl.ds(rbase, R_TILE)])

    return k(x_pad, e3)


def _tc_combine(sums, degs, x_pad, Wlt, Wrt, Wdt, b_l, b_d):
    BN = 1024
    grid = (NPAD // BN,)

    def body(a_ref, d_ref, x_ref, wl_ref, wr_ref, wd_ref, bl_ref, bd_ref,
             o_ref):
        afwd = a_ref[0]
        arev = a_ref[1]
        din = d_ref[0]
        dout = d_ref[1]
        xb = x_ref[...]
        agg = afwd / jnp.maximum(din, 1.0)
        hp = jax.lax.Precision.HIGHEST
        h = jnp.maximum(jnp.dot(agg, wl_ref[...], precision=hp)
                        + jnp.dot(xb, wr_ref[...], precision=hp)
                        + bl_ref[...], 0.0)
        o_ref[...] = (h + jnp.dot(arev - dout * xb, wd_ref[...], precision=hp)
                      + dout * bd_ref[...])

    return pl.pallas_call(
        body,
        grid=grid,
        in_specs=[
            pl.BlockSpec((2, BN, D), lambda i: (0, i, 0)),
            pl.BlockSpec((2, BN, 1), lambda i: (0, i, 0)),
            pl.BlockSpec((BN, D), lambda i: (i, 0)),
            pl.BlockSpec((D, D), lambda i: (0, 0)),
            pl.BlockSpec((D, D), lambda i: (0, 0)),
            pl.BlockSpec((D, D), lambda i: (0, 0)),
            pl.BlockSpec((1, D), lambda i: (0, 0)),
            pl.BlockSpec((1, D), lambda i: (0, 0)),
        ],
        out_specs=pl.BlockSpec((BN, D), lambda i: (i, 0)),
        out_shape=jax.ShapeDtypeStruct((NPAD, D), jnp.float32),
    )(sums, degs, x_pad, Wlt, Wrt, Wdt, b_l, b_d)


def kernel(x, edge_index, W_l, b_l, W_r, W_d, b_d):
    x = x.astype(jnp.float32)
    x_pad = jnp.concatenate(
        [x, jnp.zeros((NPAD - N, D), jnp.float32)], axis=0)
    e = edge_index.astype(jnp.int32)
    e_pad = jnp.concatenate(
        [e, jnp.full((2, E_PAD - E), N, jnp.int32)], axis=1)
    e3 = e_pad.reshape(2, NSUB * NCH, CH, B)

    sums, degs = _sc_segment_sums(x_pad, e3)
    out = _tc_combine(sums, degs[:, :, None], x_pad,
                      W_l.T, W_r.T, W_d.T, b_l[None, :], b_d[None, :])
    return out[:N]


# speedup vs baseline: 1.0595x; 1.0595x over previous
import functools

import jax
import jax.numpy as jnp
from jax import lax
from jax.experimental import pallas as pl
from jax.experimental.pallas import tpu as pltpu
from jax.experimental.pallas import tpu_sc as plsc

N = 10000
E = 320000
D = 128
NSUB = 16
B = 128
CH = 8
NCH = 20
NB = CH * NCH
E_PAD = NSUB * NB * B
NPAD = 10240
R_TILE = NPAD // NSUB


def _sc_segment_sums(x_pad, e_pad):
    mesh = plsc.VectorSubcoreMesh(core_axis_name="c", subcore_axis_name="s")

    @functools.partial(
        pl.kernel,
        mesh=mesh,
        out_type=(jax.ShapeDtypeStruct((2, NPAD, D), jnp.float32),
                  jax.ShapeDtypeStruct((2, NPAD), jnp.float32)),
        scratch_types=[
            pltpu.VMEM((B,), jnp.int32),
            pltpu.VMEM((B,), jnp.int32),
            pltpu.VMEM((B,), jnp.int32),
            pltpu.VMEM((B,), jnp.int32),
            pltpu.VMEM((B, D), jnp.float32),
            pltpu.VMEM((B, D), jnp.float32),
            pltpu.VMEM((B,), jnp.float32),
            pltpu.VMEM((R_TILE,), jnp.float32),
            pltpu.VMEM_SHARED((NPAD, D), jnp.float32),
            pltpu.VMEM_SHARED((NPAD,), jnp.float32),
            pltpu.SemaphoreType.DMA,
            pltpu.SemaphoreType.DMA,
        ],
    )
    def k(x_hbm, e_hbm, out_hbm, deg_hbm,
          g0, s0, g1, s1, r0, r1, ones, zed, acc, dacc, sem0, sem1):
        c = lax.axis_index("c")
        s = lax.axis_index("s")
        zvec = jnp.zeros((16,), jnp.float32)
        ovec = jnp.ones((16,), jnp.float32)

        def zrow(i, carry):
            for kk in range(D // 16):
                r0[i, pl.ds(kk * 16, 16)] = zvec
            return carry
        lax.fori_loop(0, B, zrow, 0)
        for j in range(B // 16):
            ones[pl.ds(j * 16, 16)] = ovec
        def zdeg(i, carry):
            zed[pl.ds(i * 16, 16)] = zvec
            return carry
        lax.fori_loop(0, R_TILE // 16, zdeg, 0)
        rbase = s * R_TILE
        for j in range(R_TILE // B):
            pltpu.sync_copy(r0, acc.at[pl.ds(rbase + j * B, B)])
        pltpu.sync_copy(zed, dacc.at[pl.ds(rbase, R_TILE)])
        plsc.subcore_barrier()

        ebase = s * NB * B

        pltpu.sync_copy(e_hbm.at[c, pl.ds(ebase, B)], g0)
        pltpu.sync_copy(e_hbm.at[1 - c, pl.ds(ebase, B)], s0)
        pltpu.async_copy(x_hbm.at[g0], r0, sem0)

        def pair(i, carry):
            base = ebase + 2 * i * B
            pltpu.sync_copy(e_hbm.at[c, pl.ds(base + B, B)], g1)
            pltpu.sync_copy(e_hbm.at[1 - c, pl.ds(base + B, B)], s1)
            pltpu.async_copy(x_hbm.at[g1], r1, sem1)
            pltpu.make_async_copy(x_hbm.at[pl.ds(0, B)], r0, sem0).wait()
            pltpu.sync_copy(r0, acc.at[s0], add=True)
            pltpu.sync_copy(ones, dacc.at[s0], add=True)
            pltpu.sync_copy(e_hbm.at[c, pl.ds(base + 2 * B, B)], g0)
            pltpu.sync_copy(e_hbm.at[1 - c, pl.ds(base + 2 * B, B)], s0)
            pltpu.async_copy(x_hbm.at[g0], r0, sem0)
            pltpu.make_async_copy(x_hbm.at[pl.ds(0, B)], r1, sem1).wait()
            pltpu.sync_copy(r1, acc.at[s1], add=True)
            pltpu.sync_copy(ones, dacc.at[s1], add=True)
            return carry

        lax.fori_loop(0, NB // 2 - 1, pair, 0)

        base = ebase + (NB - 1) * B
        pltpu.sync_copy(e_hbm.at[c, pl.ds(base, B)], g1)
        pltpu.sync_copy(e_hbm.at[1 - c, pl.ds(base, B)], s1)
        pltpu.async_copy(x_hbm.at[g1], r1, sem1)
        pltpu.make_async_copy(x_hbm.at[pl.ds(0, B)], r0, sem0).wait()
        pltpu.sync_copy(r0, acc.at[s0], add=True)
        pltpu.sync_copy(ones, dacc.at[s0], add=True)
        pltpu.make_async_copy(x_hbm.at[pl.ds(0, B)], r1, sem1).wait()
        pltpu.sync_copy(r1, acc.at[s1], add=True)
        pltpu.sync_copy(ones, dacc.at[s1], add=True)
        plsc.subcore_barrier()

        pltpu.sync_copy(acc.at[pl.ds(rbase, R_TILE)],
                        out_hbm.at[c, pl.ds(rbase, R_TILE)])
        pltpu.sync_copy(dacc.at[pl.ds(rbase, R_TILE)],
                        deg_hbm.at[c, pl.ds(rbase, R_TILE)])

    return k(x_pad, e_pad)


def _tc_combine(sums, degs, x_pad, Wlt, Wrt, Wdt, b_l, b_d):
    BN = 1024
    grid = (NPAD // BN,)

    def body(a_ref, d_ref, x_ref, wl_ref, wr_ref, wd_ref, bl_ref, bd_ref,
             o_ref):
        afwd = a_ref[0]
        arev = a_ref[1]
        din = d_ref[0]
        dout = d_ref[1]
        xb = x_ref[...]
        agg = afwd / jnp.maximum(din, 1.0)
        hp = jax.lax.Precision.HIGHEST
        h = jnp.maximum(jnp.dot(agg, wl_ref[...], precision=hp)
                        + jnp.dot(xb, wr_ref[...], precision=hp)
                        + bl_ref[...], 0.0)
        o_ref[...] = (h + jnp.dot(arev - dout * xb, wd_ref[...], precision=hp)
                      + dout * bd_ref[...])

    return pl.pallas_call(
        body,
        grid=grid,
        in_specs=[
            pl.BlockSpec((2, BN, D), lambda i: (0, i, 0)),
            pl.BlockSpec((2, BN, 1), lambda i: (0, i, 0)),
            pl.BlockSpec((BN, D), lambda i: (i, 0)),
            pl.BlockSpec((D, D), lambda i: (0, 0)),
            pl.BlockSpec((D, D), lambda i: (0, 0)),
            pl.BlockSpec((D, D), lambda i: (0, 0)),
            pl.BlockSpec((1, D), lambda i: (0, 0)),
            pl.BlockSpec((1, D), lambda i: (0, 0)),
        ],
        out_specs=pl.BlockSpec((BN, D), lambda i: (i, 0)),
        out_shape=jax.ShapeDtypeStruct((NPAD, D), jnp.float32),
    )(sums, degs, x_pad, Wlt, Wrt, Wdt, b_l, b_d)


def kernel(x, edge_index, W_l, b_l, W_r, W_d, b_d):
    x = x.astype(jnp.float32)
    x_pad = jnp.concatenate(
        [x, jnp.zeros((NPAD - N, D), jnp.float32)], axis=0)
    e = edge_index.astype(jnp.int32)
    e_pad = jnp.concatenate(
        [e, jnp.full((2, E_PAD - E), N, jnp.int32)], axis=1)
    sums, degs = _sc_segment_sums(x_pad, e_pad)
    out = _tc_combine(sums, degs[:, :, None], x_pad,
                      W_l.T, W_r.T, W_d.T, b_l[None, :], b_d[None, :])
    return out[:N]


# speedup vs baseline: 1.5047x; 1.4202x over previous
import functools

import jax
import jax.numpy as jnp
from jax import lax
from jax.experimental import pallas as pl
from jax.experimental.pallas import tpu as pltpu
from jax.experimental.pallas import tpu_sc as plsc

N = 10000
E = 320000
D = 128
NSUB = 16
B = 128
NB = 157
E_PAD = NSUB * NB * B
NPAD = 10240
R_TILE = NPAD // NSUB


def _sc_segment_sums(x_pad, e_pad):
    mesh = plsc.VectorSubcoreMesh(core_axis_name="c", subcore_axis_name="s")

    @functools.partial(
        pl.kernel,
        mesh=mesh,
        out_type=(jax.ShapeDtypeStruct((2, NPAD, D), jnp.float32),
                  jax.ShapeDtypeStruct((2, NPAD), jnp.float32)),
        scratch_types=[
            pltpu.VMEM((B,), jnp.int32),
            pltpu.VMEM((B,), jnp.int32),
            pltpu.VMEM((B, D), jnp.float32),
            pltpu.VMEM((B,), jnp.float32),
            pltpu.VMEM((R_TILE,), jnp.float32),
            pltpu.VMEM_SHARED((NPAD, D), jnp.float32),
            pltpu.VMEM_SHARED((NPAD,), jnp.float32),
        ],
    )
    def k(x_hbm, e_hbm, out_hbm, deg_hbm,
          gidx, sidx, rows, ones, zed, acc, dacc):
        c = lax.axis_index("c")
        s = lax.axis_index("s")
        zvec = jnp.zeros((16,), jnp.float32)
        ovec = jnp.ones((16,), jnp.float32)

        def zrow(i, carry):
            for kk in range(D // 16):
                rows[i, pl.ds(kk * 16, 16)] = zvec
            return carry
        lax.fori_loop(0, B, zrow, 0)
        for j in range(B // 16):
            ones[pl.ds(j * 16, 16)] = ovec
        def zdeg(i, carry):
            zed[pl.ds(i * 16, 16)] = zvec
            return carry
        lax.fori_loop(0, R_TILE // 16, zdeg, 0)
        rbase = s * R_TILE
        for j in range(R_TILE // B):
            pltpu.sync_copy(rows, acc.at[pl.ds(rbase + j * B, B)])
        pltpu.sync_copy(zed, dacc.at[pl.ds(rbase, R_TILE)])
        plsc.subcore_barrier()

        ebase = s * NB * B

        def body(i, carry):
            base = ebase + i * B
            pltpu.sync_copy(e_hbm.at[c, pl.ds(base, B)], gidx)
            pltpu.sync_copy(e_hbm.at[1 - c, pl.ds(base, B)], sidx)
            pltpu.sync_copy(x_hbm.at[gidx], rows)
            pltpu.sync_copy(rows, acc.at[sidx], add=True)
            pltpu.sync_copy(ones, dacc.at[sidx], add=True)
            return carry

        lax.fori_loop(0, NB, body, 0)
        plsc.subcore_barrier()

        pltpu.sync_copy(acc.at[pl.ds(rbase, R_TILE)],
                        out_hbm.at[c, pl.ds(rbase, R_TILE)])
        pltpu.sync_copy(dacc.at[pl.ds(rbase, R_TILE)],
                        deg_hbm.at[c, pl.ds(rbase, R_TILE)])

    return k(x_pad, e_pad)


def _tc_combine(sums, degs, x_pad, Wlt, Wrt, Wdt, b_l, b_d):
    BN = 1024
    grid = (NPAD // BN,)

    def body(a_ref, d_ref, x_ref, wl_ref, wr_ref, wd_ref, bl_ref, bd_ref,
             o_ref):
        afwd = a_ref[0]
        arev = a_ref[1]
        din = d_ref[0]
        dout = d_ref[1]
        xb = x_ref[...]
        agg = afwd / jnp.maximum(din, 1.0)
        hp = jax.lax.Precision.HIGHEST
        h = jnp.maximum(jnp.dot(agg, wl_ref[...], precision=hp)
                        + jnp.dot(xb, wr_ref[...], precision=hp)
                        + bl_ref[...], 0.0)
        o_ref[...] = (h + jnp.dot(arev - dout * xb, wd_ref[...], precision=hp)
                      + dout * bd_ref[...])

    return pl.pallas_call(
        body,
        grid=grid,
        in_specs=[
            pl.BlockSpec((2, BN, D), lambda i: (0, i, 0)),
            pl.BlockSpec((2, BN, 1), lambda i: (0, i, 0)),
            pl.BlockSpec((BN, D), lambda i: (i, 0)),
            pl.BlockSpec((D, D), lambda i: (0, 0)),
            pl.BlockSpec((D, D), lambda i: (0, 0)),
            pl.BlockSpec((D, D), lambda i: (0, 0)),
            pl.BlockSpec((1, D), lambda i: (0, 0)),
            pl.BlockSpec((1, D), lambda i: (0, 0)),
        ],
        out_specs=pl.BlockSpec((BN, D), lambda i: (i, 0)),
        out_shape=jax.ShapeDtypeStruct((NPAD, D), jnp.float32),
    )(sums, degs, x_pad, Wlt, Wrt, Wdt, b_l, b_d)


def kernel(x, edge_index, W_l, b_l, W_r, W_d, b_d):
    x = x.astype(jnp.float32)
    x_pad = jnp.concatenate(
        [x, jnp.zeros((NPAD - N, D), jnp.float32)], axis=0)
    e = edge_index.astype(jnp.int32)
    e_pad = jnp.concatenate(
        [e, jnp.full((2, E_PAD - E), N, jnp.int32)], axis=1)
    sums, degs = _sc_segment_sums(x_pad, e_pad)
    out = _tc_combine(sums, degs[:, :, None], x_pad,
                      W_l.T, W_r.T, W_d.T, b_l[None, :], b_d[None, :])
    return out[:N]
